# baseline (speedup 1.0000x reference)
import functools

import jax
import jax.numpy as jnp
from jax import lax
from jax.experimental import pallas as pl
from jax.experimental.pallas import tpu as pltpu
from jax.experimental.pallas import tpu_sc as plsc

N = 128
FLAT = N * N
L = 16


def _sc_body(nb, e, pp_hbm, tt_hbm, w_hbm, out_hbm,
             ppm, tvm, wvm, buf0, buf1, sem0, sem1):
    wid = lax.axis_index("s") * 2 + lax.axis_index("c")
    base = wid * nb
    chunks = e // L
    bufs = (buf0, buf1)
    sems = (sem0, sem1)

    pltpu.sync_copy(pp_hbm.at[pl.ds(base * e * 2, nb * e * 2)], ppm)
    pltpu.sync_copy(tt_hbm.at[pl.ds(base * e, nb * e)], tvm)
    pltpu.sync_copy(w_hbm, wvm)

    zeros = jnp.zeros((L,), jnp.float32)
    iota2 = lax.iota(jnp.int32, L) * 2

    def zero_init(i, carry):
        s = i * (8 * L)
        for buf in bufs:
            for u in range(8):
                buf[pl.ds(s + u * L, L)] = zeros
        return carry

    lax.fori_loop(0, FLAT // (8 * L), zero_init, 0)

    def scatter_batch(k, buf):
        off = k * e

        def body(c, carry):
            s = off + c * L
            pi = s * 2 + iota2
            iv = plsc.load_gather(ppm, [pi]) + 1
            jv = plsc.load_gather(ppm, [pi + 1]) + 1
            wv = plsc.load_gather(wvm, [tvm[pl.ds(s, L)]])
            plsc.store_scatter(buf, [iv * N + jv], wv)
            plsc.store_scatter(buf, [jv * N + iv], wv)
            return carry

        lax.fori_loop(0, chunks, body, 0)

    def clear_batch(k, buf):
        off = k * e

        def body(c, carry):
            s = off + c * L
            pi = s * 2 + iota2
            iv = plsc.load_gather(ppm, [pi]) + 1
            jv = plsc.load_gather(ppm, [pi + 1]) + 1
            plsc.store_scatter(buf, [iv * N + jv], zeros)
            plsc.store_scatter(buf, [jv * N + iv], zeros)
            return carry

        lax.fori_loop(0, chunks, body, 0)

    inflight = [None, None]
    for k in range(nb):
        slot = k % 2
        buf = bufs[slot]
        if inflight[slot] is not None:
            dma, kprev = inflight[slot]
            dma.wait()
            clear_batch(kprev, buf)
        scatter_batch(k, buf)
        dma = pltpu.async_copy(buf, out_hbm.at[pl.ds((base + k) * FLAT, FLAT)],
                               sems[slot])
        inflight[slot] = (dma, k)
    for slot in range(2):
        if inflight[slot] is not None:
            inflight[slot][0].wait()


def kernel(bond_idx, bond_type_idx, num_nodes, batch_size, bond_weights):
    b, e = bond_type_idx.shape
    nw = 32
    nb = b // nw

    pp = bond_idx.reshape(-1)
    tt = bond_type_idx.reshape(-1)
    ww = bond_weights.astype(jnp.float32)

    mesh = plsc.VectorSubcoreMesh(core_axis_name="c", subcore_axis_name="s")
    run = pl.kernel(
        functools.partial(_sc_body, nb, e),
        out_type=jax.ShapeDtypeStruct((b * FLAT,), jnp.float32),
        mesh=mesh,
        compiler_params=pltpu.CompilerParams(needs_layout_passes=False),
        scratch_types=[
            pltpu.VMEM((nb * e * 2,), jnp.int32),
            pltpu.VMEM((nb * e,), jnp.int32),
            pltpu.VMEM((bond_weights.shape[0],), jnp.float32),
            pltpu.VMEM((FLAT,), jnp.float32),
            pltpu.VMEM((FLAT,), jnp.float32),
            pltpu.SemaphoreType.DMA,
            pltpu.SemaphoreType.DMA,
        ],
    )
    out = run(pp, tt, ww)
    return out.reshape(b, N, N)


# speedup vs baseline: 2.1188x; 2.1188x over previous
import functools

import jax
import jax.numpy as jnp
from jax import lax
from jax.experimental import pallas as pl
from jax.experimental.pallas import tpu as pltpu
from jax.experimental.pallas import tpu_sc as plsc

N = 128
FLAT = N * N
L = 16


def _sc_body(nb, e, ii_hbm, jj_hbm, tt_hbm, w_hbm, out_hbm,
             ivm, jvm, tvm, wvm, buf0, buf1, sem0, sem1):
    wid = lax.axis_index("s") * 2 + lax.axis_index("c")
    base = wid * nb
    chunks = e // L
    bufs = (buf0, buf1)
    sems = (sem0, sem1)

    pltpu.sync_copy(ii_hbm.at[pl.ds(base * e, nb * e)], ivm)
    pltpu.sync_copy(jj_hbm.at[pl.ds(base * e, nb * e)], jvm)
    pltpu.sync_copy(tt_hbm.at[pl.ds(base * e, nb * e)], tvm)
    pltpu.sync_copy(w_hbm, wvm)

    zeros = jnp.zeros((L,), jnp.float32)

    def zero_init(i, carry):
        s = i * (8 * L)
        for buf in bufs:
            for u in range(8):
                buf[pl.ds(s + u * L, L)] = zeros
        return carry

    lax.fori_loop(0, FLAT // (8 * L), zero_init, 0)

    def scatter_batch(k, buf):
        off = k * e

        def body(c, carry):
            s = off + c * L
            iv = ivm[pl.ds(s, L)] + 1
            jv = jvm[pl.ds(s, L)] + 1
            wv = plsc.load_gather(wvm, [tvm[pl.ds(s, L)]])
            plsc.store_scatter(buf, [iv * N + jv], wv)
            plsc.store_scatter(buf, [jv * N + iv], wv)
            return carry

        lax.fori_loop(0, chunks, body, 0)

    def clear_batch(k, buf):
        off = k * e

        def body(c, carry):
            s = off + c * L
            iv = ivm[pl.ds(s, L)] + 1
            jv = jvm[pl.ds(s, L)] + 1
            plsc.store_scatter(buf, [iv * N + jv], zeros)
            plsc.store_scatter(buf, [jv * N + iv], zeros)
            return carry

        lax.fori_loop(0, chunks, body, 0)

    inflight = [None, None]
    for k in range(nb):
        slot = k % 2
        buf = bufs[slot]
        if inflight[slot] is not None:
            dma, kprev = inflight[slot]
            dma.wait()
            clear_batch(kprev, buf)
        scatter_batch(k, buf)
        dma = pltpu.async_copy(buf, out_hbm.at[pl.ds((base + k) * FLAT, FLAT)],
                               sems[slot])
        inflight[slot] = (dma, k)
    for slot in range(2):
        if inflight[slot] is not None:
            inflight[slot][0].wait()


def kernel(bond_idx, bond_type_idx, num_nodes, batch_size, bond_weights):
    b, e = bond_type_idx.shape
    nw = 32
    nb = b // nw

    ii = bond_idx[..., 0].reshape(-1)
    jj = bond_idx[..., 1].reshape(-1)
    tt = bond_type_idx.reshape(-1)
    w16 = jnp.pad(bond_weights.astype(jnp.float32), (0, L - bond_weights.shape[0]))

    mesh = plsc.VectorSubcoreMesh(core_axis_name="c", subcore_axis_name="s")
    run = pl.kernel(
        functools.partial(_sc_body, nb, e),
        out_type=jax.ShapeDtypeStruct((b * FLAT,), jnp.float32),
        mesh=mesh,
        compiler_params=pltpu.CompilerParams(needs_layout_passes=False),
        scratch_types=[
            pltpu.VMEM((nb * e,), jnp.int32),
            pltpu.VMEM((nb * e,), jnp.int32),
            pltpu.VMEM((nb * e,), jnp.int32),
            pltpu.VMEM((L,), jnp.float32),
            pltpu.VMEM((FLAT,), jnp.float32),
            pltpu.VMEM((FLAT,), jnp.float32),
            pltpu.SemaphoreType.DMA,
            pltpu.SemaphoreType.DMA,
        ],
    )
    out = run(ii, jj, tt, w16)
    return out.reshape(b, N, N)


# speedup vs baseline: 2.3531x; 1.1106x over previous
import functools

import jax
import jax.numpy as jnp
from jax import lax
from jax.experimental import pallas as pl
from jax.experimental.pallas import tpu as pltpu
from jax.experimental.pallas import tpu_sc as plsc

N = 128
FLAT = N * N
L = 16


def _sc_body(nb, e, ii_hbm, jj_hbm, tt_hbm, w_hbm, out_hbm,
             ivm, jvm, tvm, wvm, buf0, buf1, sem0, sem1):
    wid = lax.axis_index("s") * 2 + lax.axis_index("c")
    base = wid * nb
    chunks = e // L
    bufs = (buf0, buf1)
    sems = (sem0, sem1)

    stage = [
        pltpu.async_copy(ii_hbm.at[pl.ds(base * e, nb * e)], ivm, sem0),
        pltpu.async_copy(jj_hbm.at[pl.ds(base * e, nb * e)], jvm, sem0),
        pltpu.async_copy(tt_hbm.at[pl.ds(base * e, nb * e)], tvm, sem0),
        pltpu.async_copy(w_hbm, wvm, sem0),
    ]

    zeros = jnp.zeros((L,), jnp.float32)

    def zero_init(i, carry):
        s = i * (8 * L)
        for buf in bufs:
            for u in range(8):
                buf[pl.ds(s + u * L, L)] = zeros
        return carry

    lax.fori_loop(0, FLAT // (8 * L), zero_init, 0)
    for d in stage:
        d.wait()

    def scatter_batch(k, buf):
        off = k * e

        def body(c, carry):
            s = off + c * L
            iv = ivm[pl.ds(s, L)] + 1
            jv = jvm[pl.ds(s, L)] + 1
            wv = plsc.load_gather(wvm, [tvm[pl.ds(s, L)]])
            plsc.store_scatter(buf, [iv * N + jv], wv)
            plsc.store_scatter(buf, [jv * N + iv], wv)
            return carry

        lax.fori_loop(0, chunks, body, 0)

    def clear_batch(k, buf):
        off = k * e

        def body(c, carry):
            s = off + c * L
            iv = ivm[pl.ds(s, L)] + 1
            jv = jvm[pl.ds(s, L)] + 1
            plsc.store_scatter(buf, [iv * N + jv], zeros)
            plsc.store_scatter(buf, [jv * N + iv], zeros)
            return carry

        lax.fori_loop(0, chunks, body, 0)

    inflight = [None, None]
    for k in range(nb):
        slot = k % 2
        buf = bufs[slot]
        if inflight[slot] is not None:
            dma, kprev = inflight[slot]
            dma.wait()
            clear_batch(kprev, buf)
        scatter_batch(k, buf)
        dma = pltpu.async_copy(buf, out_hbm.at[pl.ds((base + k) * FLAT, FLAT)],
                               sems[slot])
        inflight[slot] = (dma, k)
    for slot in range(2):
        if inflight[slot] is not None:
            inflight[slot][0].wait()


def kernel(bond_idx, bond_type_idx, num_nodes, batch_size, bond_weights):
    b, e = bond_type_idx.shape
    nw = 32
    nb = b // nw

    ii = bond_idx[..., 0].reshape(-1)
    jj = bond_idx[..., 1].reshape(-1)
    tt = bond_type_idx.reshape(-1)
    ww = bond_weights.astype(jnp.float32)

    mesh = plsc.VectorSubcoreMesh(core_axis_name="c", subcore_axis_name="s")
    run = pl.kernel(
        functools.partial(_sc_body, nb, e),
        out_type=jax.ShapeDtypeStruct((b * FLAT,), jnp.float32),
        mesh=mesh,
        compiler_params=pltpu.CompilerParams(needs_layout_passes=False),
        scratch_types=[
            pltpu.VMEM((nb * e,), jnp.int32),
            pltpu.VMEM((nb * e,), jnp.int32),
            pltpu.VMEM((nb * e,), jnp.int32),
            pltpu.VMEM((bond_weights.shape[0],), jnp.float32),
            pltpu.VMEM((FLAT,), jnp.float32),
            pltpu.VMEM((FLAT,), jnp.float32),
            pltpu.SemaphoreType.DMA,
            pltpu.SemaphoreType.DMA,
        ],
    )
    out = run(ii, jj, tt, ww)
    return out.reshape(b, N, N)


# speedup vs baseline: 2.4284x; 1.0320x over previous
import functools

import jax
import jax.numpy as jnp
from jax import lax
from jax.experimental import pallas as pl
from jax.experimental.pallas import tpu as pltpu
from jax.experimental.pallas import tpu_sc as plsc

N = 128
FLAT = N * N
L = 16


def _sc_body(nb, e, be, pk_hbm, w_hbm, out_hbm,
             ivm, jvm, tvm, wvm, buf0, buf1, sem0, sem1):
    wid = lax.axis_index("s") * 2 + lax.axis_index("c")
    base = wid * nb
    chunks = e // L
    bufs = (buf0, buf1)
    sems = (sem0, sem1)

    stage = [
        pltpu.async_copy(pk_hbm.at[pl.ds(base * e, nb * e)], ivm, sem0),
        pltpu.async_copy(pk_hbm.at[pl.ds(be + base * e, nb * e)], jvm, sem0),
        pltpu.async_copy(pk_hbm.at[pl.ds(2 * be + base * e, nb * e)], tvm, sem0),
        pltpu.async_copy(w_hbm, wvm, sem0),
    ]

    zeros = jnp.zeros((L,), jnp.float32)

    def zero_init(i, carry):
        s = i * (8 * L)
        for buf in bufs:
            for u in range(8):
                buf[pl.ds(s + u * L, L)] = zeros
        return carry

    lax.fori_loop(0, FLAT // (8 * L), zero_init, 0)
    for d in stage:
        d.wait()

    def scatter_batch(k, buf):
        off = k * e

        def body(c, carry):
            s = off + c * L
            iv = ivm[pl.ds(s, L)] + 1
            jv = jvm[pl.ds(s, L)] + 1
            wv = plsc.load_gather(wvm, [tvm[pl.ds(s, L)]])
            plsc.store_scatter(buf, [iv * N + jv], wv)
            plsc.store_scatter(buf, [jv * N + iv], wv)
            return carry

        lax.fori_loop(0, chunks, body, 0)

    def clear_batch(k, buf):
        off = k * e

        def body(c, carry):
            s = off + c * L
            iv = ivm[pl.ds(s, L)] + 1
            jv = jvm[pl.ds(s, L)] + 1
            plsc.store_scatter(buf, [iv * N + jv], zeros)
            plsc.store_scatter(buf, [jv * N + iv], zeros)
            return carry

        lax.fori_loop(0, chunks, body, 0)

    inflight = [None, None]
    for k in range(nb):
        slot = k % 2
        buf = bufs[slot]
        if inflight[slot] is not None:
            dma, kprev = inflight[slot]
            dma.wait()
            clear_batch(kprev, buf)
        scatter_batch(k, buf)
        dma = pltpu.async_copy(buf, out_hbm.at[pl.ds((base + k) * FLAT, FLAT)],
                               sems[slot])
        inflight[slot] = (dma, k)
    for slot in range(2):
        if inflight[slot] is not None:
            inflight[slot][0].wait()


def kernel(bond_idx, bond_type_idx, num_nodes, batch_size, bond_weights):
    b, e = bond_type_idx.shape
    nw = 32
    nb = b // nw

    pk = jnp.concatenate([bond_idx[..., 0].reshape(-1),
                          bond_idx[..., 1].reshape(-1),
                          bond_type_idx.reshape(-1)])
    ww = bond_weights.astype(jnp.float32)

    mesh = plsc.VectorSubcoreMesh(core_axis_name="c", subcore_axis_name="s")
    run = pl.kernel(
        functools.partial(_sc_body, nb, e, b * e),
        out_type=jax.ShapeDtypeStruct((b * FLAT,), jnp.float32),
        mesh=mesh,
        compiler_params=pltpu.CompilerParams(needs_layout_passes=False),
        scratch_types=[
            pltpu.VMEM((nb * e,), jnp.int32),
            pltpu.VMEM((nb * e,), jnp.int32),
            pltpu.VMEM((nb * e,), jnp.int32),
            pltpu.VMEM((bond_weights.shape[0],), jnp.float32),
            pltpu.VMEM((FLAT,), jnp.float32),
            pltpu.VMEM((FLAT,), jnp.float32),
            pltpu.SemaphoreType.DMA,
            pltpu.SemaphoreType.DMA,
        ],
    )
    out = run(pk, ww)
    return out.reshape(b, N, N)


# speedup vs baseline: 2.4557x; 1.0113x over previous
import functools

import jax
import jax.numpy as jnp
from jax import lax
from jax.experimental import pallas as pl
from jax.experimental.pallas import tpu as pltpu
from jax.experimental.pallas import tpu_sc as plsc

N = 128
FLAT = N * N
L = 16


def _sc_body(nb, e, be, pk_hbm, w_hbm, out_hbm,
             ivm, jvm, tvm, wvm, buf0, buf1, sem0, sem1):
    wid = lax.axis_index("s") * 2 + lax.axis_index("c")
    base = wid * nb
    chunks = e // L
    bufs = (buf0, buf1)
    sems = (sem0, sem1)

    stage = [
        pltpu.async_copy(pk_hbm.at[pl.ds(base * e, nb * e)], ivm, sem0),
        pltpu.async_copy(pk_hbm.at[pl.ds(be + base * e, nb * e)], jvm, sem0),
        pltpu.async_copy(pk_hbm.at[pl.ds(2 * be + base * e, nb * e)], tvm, sem0),
        pltpu.async_copy(w_hbm, wvm, sem0),
    ]

    zeros = jnp.zeros((L,), jnp.float32)

    def zero_init(i, carry):
        s = i * (8 * L)
        for buf in bufs:
            for u in range(8):
                buf[pl.ds(s + u * L, L)] = zeros
        return carry

    lax.fori_loop(0, FLAT // (8 * L), zero_init, 0)
    for d in stage:
        d.wait()

    def scatter_batch(k, buf):
        off = k * e

        def body(c, carry):
            s = off + c * L
            iv = ivm[pl.ds(s, L)] + 1
            jv = jvm[pl.ds(s, L)] + 1
            wv = plsc.load_gather(wvm, [tvm[pl.ds(s, L)]])
            plsc.store_scatter(buf, [iv * N + jv], wv)
            plsc.store_scatter(buf, [jv * N + iv], wv)
            return carry

        lax.fori_loop(0, chunks, body, 0)

    def clear_batch(k, buf):
        off = k * e

        def body(c, carry):
            s = off + c * L
            iv = ivm[pl.ds(s, L)] + 1
            jv = jvm[pl.ds(s, L)] + 1
            plsc.store_scatter(buf, [iv * N + jv], zeros)
            plsc.store_scatter(buf, [jv * N + iv], zeros)
            return carry

        lax.fori_loop(0, chunks, body, 0)

    def out_slice(k):
        return out_hbm.at[pl.ds((base + k) * FLAT, FLAT)]

    def group(g, carry):
        for slot in range(2):
            k = g * 2 + slot
            buf, sem = bufs[slot], sems[slot]

            @pl.when(g > 0)
            def _():
                pltpu.make_async_copy(buf, out_slice(k - 2), sem).wait()
                clear_batch(k - 2, buf)

            scatter_batch(k, buf)
            pltpu.async_copy(buf, out_slice(k), sem)
        return carry

    lax.fori_loop(0, nb // 2, group, 0)
    for slot in range(2):
        k = nb - 2 + slot
        pltpu.make_async_copy(bufs[slot], out_slice(k), sems[slot]).wait()


def kernel(bond_idx, bond_type_idx, num_nodes, batch_size, bond_weights):
    b, e = bond_type_idx.shape
    nw = 32
    nb = b // nw

    pk = jnp.concatenate([bond_idx[..., 0].reshape(-1),
                          bond_idx[..., 1].reshape(-1),
                          bond_type_idx.reshape(-1)])
    ww = bond_weights

    mesh = plsc.VectorSubcoreMesh(core_axis_name="c", subcore_axis_name="s")
    run = pl.kernel(
        functools.partial(_sc_body, nb, e, b * e),
        out_type=jax.ShapeDtypeStruct((b * FLAT,), jnp.float32),
        mesh=mesh,
        compiler_params=pltpu.CompilerParams(needs_layout_passes=False),
        scratch_types=[
            pltpu.VMEM((nb * e,), jnp.int32),
            pltpu.VMEM((nb * e,), jnp.int32),
            pltpu.VMEM((nb * e,), jnp.int32),
            pltpu.VMEM((bond_weights.shape[0],), jnp.float32),
            pltpu.VMEM((FLAT,), jnp.float32),
            pltpu.VMEM((FLAT,), jnp.float32),
            pltpu.SemaphoreType.DMA,
            pltpu.SemaphoreType.DMA,
        ],
    )
    out = run(pk, ww)
    return out.reshape(b, N, N)
